# Initial kernel scaffold; baseline (speedup 1.0000x reference)
import jax
import jax.numpy as jnp
from jax.experimental import pallas as pl


def kernel(x, edge_index, edge_attr, W1, b1, W2, b2, root, bias, gamma, beta):
    raise NotImplementedError("write your pallas kernel here")


# speedup vs baseline: 2.0319x; 2.0319x over previous
import functools
import math

import jax
import jax.numpy as jnp
from jax import lax
from jax.experimental import pallas as pl
from jax.experimental.pallas import tpu as pltpu
from jax.experimental.pallas import tpu_sc as plsc

_EPS = 1e-5
_NC = 2
_NS = 16
_NW = _NC * _NS


def _pick_mac(gpw):
    for m in (40, 32, 24, 16, 8):
        if gpw % m == 0:
            return m
    return 8


def _sc_gather(x, src2d):
    G = src2d.shape[0]
    D = x.shape[1]
    gpw = G // _NW
    mac = _pick_mac(gpw)
    nmac = gpw // mac
    mesh = plsc.VectorSubcoreMesh(core_axis_name="c", subcore_axis_name="s")

    @functools.partial(
        pl.kernel,
        mesh=mesh,
        out_type=jax.ShapeDtypeStruct((G, 128, D), jnp.float32),
        scratch_types=[
            pltpu.VMEM((mac, 128), jnp.int32),
            pltpu.VMEM((mac, 128, D), jnp.float32),
            pltpu.SemaphoreType.DMA,
        ],
        compiler_params=pltpu.CompilerParams(use_tc_tiling_on_sc=False),
    )
    def k(x_hbm, src_hbm, out_hbm, idx_v, rows_v, sem):
        wid = lax.axis_index("s") * _NC + lax.axis_index("c")
        g0 = wid * gpw
        for m in range(nmac):
            gb = g0 + m * mac
            pltpu.sync_copy(src_hbm.at[pl.ds(gb, mac)], idx_v)
            cps = [
                pltpu.async_copy(x_hbm.at[idx_v.at[r]], rows_v.at[r], sem)
                for r in range(mac)
            ]
            for c in cps:
                c.wait()
            pltpu.sync_copy(rows_v, out_hbm.at[pl.ds(gb, mac)])

    return k(x, src2d)


def _sc_scatter(msg3d, dst2d, zero_nd, zero_n, ones128, half, h1):
    G = dst2d.shape[0]
    D = msg3d.shape[2]
    mac = 8
    nmac = G // (_NS * mac)
    rpt = h1 // _NS
    trash = h1 - 1
    mesh = plsc.VectorSubcoreMesh(core_axis_name="c", subcore_axis_name="s")

    @functools.partial(
        pl.kernel,
        mesh=mesh,
        out_type=(
            jax.ShapeDtypeStruct((_NC * h1, D), jnp.float32),
            jax.ShapeDtypeStruct((_NC * h1,), jnp.float32),
        ),
        scratch_types=[
            pltpu.VMEM((mac, 128), jnp.int32),
            pltpu.VMEM((mac, 128), jnp.int32),
            pltpu.VMEM((mac, 128, D), jnp.float32),
            pltpu.VMEM((128,), jnp.float32),
            pltpu.VMEM_SHARED((h1, D), jnp.float32),
            pltpu.VMEM_SHARED((h1,), jnp.float32),
            pltpu.SemaphoreType.DMA,
        ],
        compiler_params=pltpu.CompilerParams(use_tc_tiling_on_sc=False),
    )
    def k(msg_hbm, dst_hbm, z2_hbm, z1_hbm, one_hbm, agg_out, cnt_out,
          idx_v, idx2_v, rows_v, ones_v, agg_sh, cnt_sh, sem):
        cid = lax.axis_index("c")
        sid = lax.axis_index("s")
        r0 = sid * rpt
        lo_bound = cid * half
        pltpu.sync_copy(z2_hbm.at[pl.ds(r0, rpt)], agg_sh.at[pl.ds(r0, rpt)])
        pltpu.sync_copy(z1_hbm.at[pl.ds(r0, rpt)], cnt_sh.at[pl.ds(r0, rpt)])
        pltpu.sync_copy(one_hbm, ones_v)
        plsc.subcore_barrier()

        def step(m, carry):
            gb = pl.multiple_of((sid * nmac + m) * mac, 8)
            pltpu.sync_copy(dst_hbm.at[pl.ds(gb, mac)], idx_v)
            pltpu.sync_copy(msg_hbm.at[pl.ds(gb, mac)], rows_v)
            for g in range(mac):
                for c16 in range(8):
                    d = idx_v[g, pl.ds(c16 * 16, 16)]
                    loc = d - lo_bound
                    ok = (loc >= 0) & (loc < half)
                    idx2_v[g, pl.ds(c16 * 16, 16)] = jnp.where(ok, loc, trash)
            for g in range(mac):
                pltpu.sync_copy(rows_v.at[g], agg_sh.at[idx2_v.at[g]],
                                add=True)
                pltpu.sync_copy(ones_v, cnt_sh.at[idx2_v.at[g]], add=True)
            return carry

        lax.fori_loop(0, nmac, step, 0)
        plsc.subcore_barrier()
        pltpu.sync_copy(agg_sh.at[pl.ds(r0, rpt)],
                        agg_out.at[pl.ds(cid * h1 + r0, rpt)])
        pltpu.sync_copy(cnt_sh.at[pl.ds(r0, rpt)],
                        cnt_out.at[pl.ds(cid * h1 + r0, rpt)])

    return k(msg3d, dst2d, zero_nd, zero_n, ones128)


def _tc_edge(ea_p, xj_p, W1T, b1c, W2T, b2c):
    Ep, DE = ea_p.shape
    D = xj_p.shape[1]
    H = W1T.shape[0]
    DD = D * D
    for B in (1024, 512, 256, 128):
        if Ep % B == 0:
            break
    nblk = Ep // B

    def body(ea_ref, xj_ref, w1_ref, b1_ref, w2_ref, b2_ref, out_ref):
        eaT = jnp.transpose(ea_ref[...])
        xjT = jnp.transpose(xj_ref[...])
        hidT = jnp.maximum(
            jnp.dot(w1_ref[...], eaT, preferred_element_type=jnp.float32)
            + b1_ref[...], 0.0)
        wT = jnp.dot(w2_ref[...], hidT,
                     preferred_element_type=jnp.float32) + b2_ref[...]
        xrep = jnp.broadcast_to(xjT[:, None, :], (D, D, B)).reshape(DD, B)
        msgT = jnp.sum((xrep * wT).reshape(D, D, B), axis=0)
        out_ref[...] = jnp.transpose(msgT)

    return pl.pallas_call(
        body,
        grid=(nblk,),
        in_specs=[
            pl.BlockSpec((B, DE), lambda i: (i, 0)),
            pl.BlockSpec((B, D), lambda i: (i, 0)),
            pl.BlockSpec((H, DE), lambda i: (0, 0)),
            pl.BlockSpec((H, 1), lambda i: (0, 0)),
            pl.BlockSpec((DD, H), lambda i: (0, 0)),
            pl.BlockSpec((DD, 1), lambda i: (0, 0)),
        ],
        out_specs=pl.BlockSpec((B, D), lambda i: (i, 0)),
        out_shape=jax.ShapeDtypeStruct((Ep, D), jnp.float32),
    )(ea_p, xj_p, W1T, b1c, W2T, b2c)


def _tc_final(xp, aggp, cntp, bd, rsel, msel, bias_t, gamma_t, beta_t, n_real):
    N8 = xp.shape[0]
    inv_n = 1.0 / float(n_real)

    def body(x_ref, a_ref, c_ref, bd_ref, r_ref, m_ref, bias_ref, g_ref,
             be_ref, o_ref):
        inv = 1.0 / jnp.maximum(c_ref[...], 1.0)
        inv128 = jnp.dot(inv, r_ref[...],
                         preferred_element_type=jnp.float32)
        h = (a_ref[...] * inv128
             + jnp.dot(x_ref[...], bd_ref[...],
                       preferred_element_type=jnp.float32)
             + bias_ref[...])
        s1 = jnp.sum(h, axis=0, keepdims=True)
        mean = jnp.dot(s1, m_ref[...],
                       preferred_element_type=jnp.float32) * inv_n
        d = h - mean
        s2 = jnp.sum(d * d, axis=0, keepdims=True)
        var = jnp.dot(s2, m_ref[...],
                      preferred_element_type=jnp.float32) * inv_n
        hn = d * lax.rsqrt(var + _EPS) * g_ref[...] + be_ref[...]
        o_ref[...] = x_ref[...] + jnp.maximum(hn, 0.0)

    return pl.pallas_call(
        body,
        out_shape=jax.ShapeDtypeStruct((N8, 128), jnp.float32),
    )(xp, aggp, cntp, bd, rsel, msel, bias_t, gamma_t, beta_t)


def kernel(x, edge_index, edge_attr, W1, b1, W2, b2, root, bias, gamma, beta):
    Nn, D = x.shape
    E, DE = edge_attr.shape
    H = W1.shape[1]

    src = edge_index[0].astype(jnp.int32)
    dst = edge_index[1].astype(jnp.int32)

    G = math.ceil(E / 128)
    G = math.ceil(G / (8 * _NW)) * (8 * _NW)
    Ep = G * 128
    pad = Ep - E
    src_p = jnp.concatenate([src, jnp.zeros((pad,), jnp.int32)]).reshape(G, 128)
    dst_p = jnp.concatenate(
        [dst, jnp.full((pad,), Nn, jnp.int32)]).reshape(G, 128)
    half = math.ceil((Nn + 1) / 2)
    h1 = math.ceil((half + 1) / 128) * 128
    ea_p = jnp.concatenate(
        [edge_attr, jnp.zeros((pad, DE), jnp.float32)], axis=0)

    xj3 = _sc_gather(x, src_p)
    xj_p = xj3.reshape(Ep, D)

    msg = _tc_edge(ea_p, xj_p, W1.T, b1.reshape(H, 1),
                   W2.T, b2.reshape(D * D, 1))

    zero_nd = jnp.zeros((h1, D), jnp.float32)
    zero_n = jnp.zeros((h1,), jnp.float32)
    ones128 = jnp.ones((128,), jnp.float32)
    agg_o, cnt_o = _sc_scatter(msg.reshape(G, 128, D), dst_p,
                               zero_nd, zero_n, ones128, half, h1)
    agg_s = jnp.concatenate([agg_o[0:half], agg_o[h1:h1 + half]])[:Nn]
    cnt_s = jnp.concatenate([cnt_o[0:half], cnt_o[h1:h1 + half]])[:Nn]

    P = 128 // D
    N8 = Nn // P
    bd = jnp.kron(jnp.eye(P, dtype=jnp.float32), root)
    rsel = jnp.kron(jnp.eye(P, dtype=jnp.float32),
                    jnp.ones((1, D), jnp.float32))
    msel = jnp.kron(jnp.ones((P, P), jnp.float32),
                    jnp.eye(D, dtype=jnp.float32))
    outp = _tc_final(x.reshape(N8, 128), agg_s.reshape(N8, 128),
                     cnt_s.reshape(N8, P), bd, rsel, msel,
                     jnp.tile(bias, P).reshape(1, 128),
                     jnp.tile(gamma, P).reshape(1, 128),
                     jnp.tile(beta, P).reshape(1, 128), Nn)
    return outp.reshape(Nn, D)


# speedup vs baseline: 3.6721x; 1.8073x over previous
import functools
import math

import jax
import jax.numpy as jnp
from jax import lax
from jax.experimental import pallas as pl
from jax.experimental.pallas import tpu as pltpu
from jax.experimental.pallas import tpu_sc as plsc

_EPS = 1e-5
_NC = 2
_NS = 16
_NW = _NC * _NS


def _pick_mac(gpw):
    for m in (40, 32, 24, 16, 8):
        if gpw % m == 0:
            return m
    return 8


def _sc_gather(x, src2d):
    G = src2d.shape[0]
    D = x.shape[1]
    gpw = G // _NW
    mac = _pick_mac(gpw)
    nmac = gpw // mac
    mesh = plsc.VectorSubcoreMesh(core_axis_name="c", subcore_axis_name="s")

    @functools.partial(
        pl.kernel,
        mesh=mesh,
        out_type=jax.ShapeDtypeStruct((G, 128, D), jnp.float32),
        scratch_types=[
            pltpu.VMEM((mac, 128), jnp.int32),
            pltpu.VMEM((mac, 128, D), jnp.float32),
            pltpu.SemaphoreType.DMA,
        ],
        compiler_params=pltpu.CompilerParams(use_tc_tiling_on_sc=False),
    )
    def k(x_hbm, src_hbm, out_hbm, idx_v, rows_v, sem):
        wid = lax.axis_index("s") * _NC + lax.axis_index("c")
        g0 = wid * gpw
        for m in range(nmac):
            gb = g0 + m * mac
            pltpu.sync_copy(src_hbm.at[pl.ds(gb, mac)], idx_v)
            cps = [
                pltpu.async_copy(x_hbm.at[idx_v.at[r]], rows_v.at[r], sem)
                for r in range(mac)
            ]
            for c in cps:
                c.wait()
            pltpu.sync_copy(rows_v, out_hbm.at[pl.ds(gb, mac)])

    return k(x, src2d)


def _sc_scatter(msg3d, dst2d, zero_nd, zero_n, ones128, h1, a1):
    G = dst2d.shape[0]
    D = msg3d.shape[2]
    mac = 8
    nmac = G // (_NS * mac)
    rpt_z = a1 // _NS
    rpt = h1 // _NS
    trash = a1 - 1
    mesh = plsc.VectorSubcoreMesh(core_axis_name="c", subcore_axis_name="s")

    @functools.partial(
        pl.kernel,
        mesh=mesh,
        out_type=(
            jax.ShapeDtypeStruct((_NC * h1, D), jnp.float32),
            jax.ShapeDtypeStruct((_NC * h1,), jnp.float32),
        ),
        scratch_types=[
            pltpu.VMEM((mac, 128), jnp.int32),
            pltpu.VMEM((mac, 128), jnp.int32),
            pltpu.VMEM((mac, 128, D), jnp.float32),
            pltpu.VMEM((128,), jnp.float32),
            pltpu.VMEM_SHARED((a1, D), jnp.float32),
            pltpu.VMEM_SHARED((a1,), jnp.float32),
            pltpu.SemaphoreType.DMA,
        ],
        compiler_params=pltpu.CompilerParams(use_tc_tiling_on_sc=False),
    )
    def k(msg_hbm, dst_hbm, z2_hbm, z1_hbm, one_hbm, agg_out, cnt_out,
          idx_v, idx2_v, rows_v, ones_v, agg_sh, cnt_sh, sem):
        cid = lax.axis_index("c")
        sid = lax.axis_index("s")
        rz0 = sid * rpt_z
        r0 = sid * rpt
        lo_bound = cid * h1
        pltpu.sync_copy(z2_hbm.at[pl.ds(rz0, rpt_z)],
                        agg_sh.at[pl.ds(rz0, rpt_z)])
        pltpu.sync_copy(z1_hbm.at[pl.ds(rz0, rpt_z)],
                        cnt_sh.at[pl.ds(rz0, rpt_z)])
        pltpu.sync_copy(one_hbm, ones_v)
        plsc.subcore_barrier()

        def step(m, carry):
            gb = pl.multiple_of((sid * nmac + m) * mac, 8)
            pltpu.sync_copy(dst_hbm.at[pl.ds(gb, mac)], idx_v)
            pltpu.sync_copy(msg_hbm.at[pl.ds(gb, mac)], rows_v)
            for g in range(mac):
                for c16 in range(8):
                    d = idx_v[g, pl.ds(c16 * 16, 16)]
                    loc = d - lo_bound
                    ok = (loc >= 0) & (loc < h1)
                    idx2_v[g, pl.ds(c16 * 16, 16)] = jnp.where(ok, loc, trash)
            for g in range(mac):
                pltpu.sync_copy(rows_v.at[g], agg_sh.at[idx2_v.at[g]],
                                add=True)
                pltpu.sync_copy(ones_v, cnt_sh.at[idx2_v.at[g]], add=True)
            return carry

        lax.fori_loop(0, nmac, step, 0)
        plsc.subcore_barrier()
        pltpu.sync_copy(agg_sh.at[pl.ds(r0, rpt)],
                        agg_out.at[pl.ds(cid * h1 + r0, rpt)])
        pltpu.sync_copy(cnt_sh.at[pl.ds(r0, rpt)],
                        cnt_out.at[pl.ds(cid * h1 + r0, rpt)])

    return k(msg3d, dst2d, zero_nd, zero_n, ones128)


def _tc_edge(ea_t, xjp, W1T, b1c, W2T, b2c, D, B):
    DE, Ep = ea_t.shape
    H = W1T.shape[0]
    DD = W2T.shape[0]
    Bp = B // 8
    nblk = Ep // B

    def body(ea_ref, xj_ref, w1_ref, b1_ref, w2_ref, b2_ref, out_ref):
        eaT = ea_ref[...]
        xpt = jnp.transpose(xj_ref[...])
        xjT = jnp.concatenate(
            [xpt[j * D:(j + 1) * D, :] for j in range(8)], axis=1)
        hidT = jnp.maximum(
            jnp.dot(w1_ref[...], eaT, preferred_element_type=jnp.float32)
            + b1_ref[...], 0.0)
        wT = jnp.dot(w2_ref[...], hidT,
                     preferred_element_type=jnp.float32) + b2_ref[...]
        xrep = jnp.broadcast_to(xjT[:, None, :], (D, D, B)).reshape(DD, B)
        msgT = jnp.sum((xrep * wT).reshape(D, D, B), axis=0)
        mstack = jnp.concatenate(
            [msgT[:, j * Bp:(j + 1) * Bp] for j in range(8)], axis=0)
        out_ref[...] = jnp.transpose(mstack)

    return pl.pallas_call(
        body,
        grid=(nblk,),
        in_specs=[
            pl.BlockSpec((DE, B), lambda i: (0, i)),
            pl.BlockSpec((Bp, 128), lambda i: (i, 0)),
            pl.BlockSpec((H, DE), lambda i: (0, 0)),
            pl.BlockSpec((H, 1), lambda i: (0, 0)),
            pl.BlockSpec((DD, H), lambda i: (0, 0)),
            pl.BlockSpec((DD, 1), lambda i: (0, 0)),
        ],
        out_specs=pl.BlockSpec((Bp, 128), lambda i: (i, 0)),
        out_shape=jax.ShapeDtypeStruct((Ep // 8, 128), jnp.float32),
    )(ea_t, xjp, W1T, b1c, W2T, b2c)


def _tc_final(xp, aggp, cntp, bd, rsel, msel, bias_t, gamma_t, beta_t, n_real):
    N8 = xp.shape[0]
    inv_n = 1.0 / float(n_real)

    def body(x_ref, a_ref, c_ref, bd_ref, r_ref, m_ref, bias_ref, g_ref,
             be_ref, o_ref):
        inv = 1.0 / jnp.maximum(c_ref[...], 1.0)
        inv128 = jnp.dot(inv, r_ref[...],
                         preferred_element_type=jnp.float32)
        h = (a_ref[...] * inv128
             + jnp.dot(x_ref[...], bd_ref[...],
                       preferred_element_type=jnp.float32)
             + bias_ref[...])
        s1 = jnp.sum(h, axis=0, keepdims=True)
        mean = jnp.dot(s1, m_ref[...],
                       preferred_element_type=jnp.float32) * inv_n
        d = h - mean
        s2 = jnp.sum(d * d, axis=0, keepdims=True)
        var = jnp.dot(s2, m_ref[...],
                      preferred_element_type=jnp.float32) * inv_n
        hn = d * lax.rsqrt(var + _EPS) * g_ref[...] + be_ref[...]
        o_ref[...] = x_ref[...] + jnp.maximum(hn, 0.0)

    return pl.pallas_call(
        body,
        out_shape=jax.ShapeDtypeStruct((N8, 128), jnp.float32),
    )(xp, aggp, cntp, bd, rsel, msel, bias_t, gamma_t, beta_t)


def kernel(x, edge_index, edge_attr, W1, b1, W2, b2, root, bias, gamma, beta):
    Nn, D = x.shape
    E, DE = edge_attr.shape
    H = W1.shape[1]

    src = edge_index[0].astype(jnp.int32)
    dst = edge_index[1].astype(jnp.int32)

    G = math.ceil(E / 128)
    G = math.ceil(G / (8 * _NW)) * (8 * _NW)
    Ep = G * 128
    pad = Ep - E
    src_p = jnp.concatenate([src, jnp.zeros((pad,), jnp.int32)]).reshape(G, 128)
    dst_p = jnp.concatenate(
        [dst, jnp.full((pad,), Nn, jnp.int32)]).reshape(G, 128)
    h1 = math.ceil((Nn + 1) / 256) * 128
    a1 = h1 + 128
    B = 2048
    Bp = B // 8
    ea_t = jnp.concatenate(
        [edge_attr.T, jnp.zeros((DE, pad), jnp.float32)], axis=1)
    ea_t = jnp.stack(
        [ea_t[:, j::8].reshape(DE, Ep // B, Bp) for j in range(8)],
        axis=2).reshape(DE, Ep)

    xj3 = _sc_gather(x, src_p)

    xjp = xj3.reshape(Ep // 8, 128)
    msgp = _tc_edge(ea_t, xjp, W1.T, b1.reshape(H, 1),
                    W2.T, b2.reshape(D * D, 1), D, B)

    zero_nd = jnp.zeros((a1, D), jnp.float32)
    zero_n = jnp.zeros((a1,), jnp.float32)
    ones128 = jnp.ones((128,), jnp.float32)
    agg_o, cnt_o = _sc_scatter(msgp.reshape(G, 128, D), dst_p,
                               zero_nd, zero_n, ones128, h1, a1)

    P = 128 // D
    N8 = Nn // P
    aggp = agg_o.reshape(2 * h1 * D // 128, 128)[:N8]
    cntp = cnt_o.reshape(2 * h1 // P, P)[:N8]
    bd = jnp.kron(jnp.eye(P, dtype=jnp.float32), root)
    rsel = jnp.kron(jnp.eye(P, dtype=jnp.float32),
                    jnp.ones((1, D), jnp.float32))
    msel = jnp.kron(jnp.ones((P, P), jnp.float32),
                    jnp.eye(D, dtype=jnp.float32))
    outp = _tc_final(x.reshape(N8, 128), aggp, cntp, bd, rsel, msel,
                     jnp.tile(bias, P).reshape(1, 128),
                     jnp.tile(gamma, P).reshape(1, 128),
                     jnp.tile(beta, P).reshape(1, 128), Nn)
    return outp.reshape(Nn, D)


# speedup vs baseline: 3.6802x; 1.0022x over previous
import functools
import math

import jax
import jax.numpy as jnp
from jax import lax
from jax.experimental import pallas as pl
from jax.experimental.pallas import tpu as pltpu
from jax.experimental.pallas import tpu_sc as plsc

_EPS = 1e-5
_NC = 2
_NS = 16
_NW = _NC * _NS


def _pick_mac(gpw):
    for m in (40, 32, 24, 16, 8):
        if gpw % m == 0:
            return m
    return 8


def _sc_gather(x, src2d):
    G = src2d.shape[0]
    D = x.shape[1]
    gpw = G // _NW
    mac = _pick_mac(gpw)
    nmac = gpw // mac
    mesh = plsc.VectorSubcoreMesh(core_axis_name="c", subcore_axis_name="s")

    @functools.partial(
        pl.kernel,
        mesh=mesh,
        out_type=jax.ShapeDtypeStruct((G, 128, D), jnp.float32),
        scratch_types=[
            pltpu.VMEM((mac, 128), jnp.int32),
            pltpu.VMEM((mac, 128, D), jnp.float32),
            pltpu.SemaphoreType.DMA,
        ],
        compiler_params=pltpu.CompilerParams(use_tc_tiling_on_sc=False),
    )
    def k(x_hbm, src_hbm, out_hbm, idx_v, rows_v, sem):
        wid = lax.axis_index("s") * _NC + lax.axis_index("c")
        g0 = wid * gpw
        for m in range(nmac):
            gb = g0 + m * mac
            pltpu.sync_copy(src_hbm.at[pl.ds(gb, mac)], idx_v)
            cps = [
                pltpu.async_copy(x_hbm.at[idx_v.at[r]], rows_v.at[r], sem)
                for r in range(mac)
            ]
            for c in cps:
                c.wait()
            pltpu.sync_copy(rows_v, out_hbm.at[pl.ds(gb, mac)])

    return k(x, src2d)


def _sc_scatter(msg3d, dst2d, zero_nd, zero_n, ones128, h1, a1):
    G = dst2d.shape[0]
    D = msg3d.shape[2]
    mac = 16
    nmac = G // (_NS * mac)
    rpt_z = a1 // _NS
    rpt = h1 // _NS
    trash = a1 - 1
    mesh = plsc.VectorSubcoreMesh(core_axis_name="c", subcore_axis_name="s")

    @functools.partial(
        pl.kernel,
        mesh=mesh,
        out_type=(
            jax.ShapeDtypeStruct((_NC * h1, D), jnp.float32),
            jax.ShapeDtypeStruct((_NC * h1,), jnp.float32),
        ),
        scratch_types=[
            pltpu.VMEM((mac, 128), jnp.int32),
            pltpu.VMEM((mac, 128), jnp.int32),
            pltpu.VMEM((mac, 128, D), jnp.float32),
            pltpu.VMEM((128,), jnp.float32),
            pltpu.VMEM_SHARED((a1, D), jnp.float32),
            pltpu.VMEM_SHARED((a1,), jnp.float32),
            pltpu.SemaphoreType.DMA,
        ],
        compiler_params=pltpu.CompilerParams(use_tc_tiling_on_sc=False),
    )
    def k(msg_hbm, dst_hbm, z2_hbm, z1_hbm, one_hbm, agg_out, cnt_out,
          idx_v, idx2_v, rows_v, ones_v, agg_sh, cnt_sh, sem):
        cid = lax.axis_index("c")
        sid = lax.axis_index("s")
        rz0 = sid * rpt_z
        r0 = sid * rpt
        lo_bound = cid * h1
        pltpu.sync_copy(z2_hbm.at[pl.ds(rz0, rpt_z)],
                        agg_sh.at[pl.ds(rz0, rpt_z)])
        pltpu.sync_copy(z1_hbm.at[pl.ds(rz0, rpt_z)],
                        cnt_sh.at[pl.ds(rz0, rpt_z)])
        pltpu.sync_copy(one_hbm, ones_v)
        plsc.subcore_barrier()

        def step(m, carry):
            gb = pl.multiple_of((sid * nmac + m) * mac, 8)
            pltpu.sync_copy(dst_hbm.at[pl.ds(gb, mac)], idx_v)
            pltpu.sync_copy(msg_hbm.at[pl.ds(gb, mac)], rows_v)
            for g in range(mac):
                for c16 in range(8):
                    d = idx_v[g, pl.ds(c16 * 16, 16)]
                    loc = d - lo_bound
                    ok = (loc >= 0) & (loc < h1)
                    idx2_v[g, pl.ds(c16 * 16, 16)] = jnp.where(ok, loc, trash)
            cps = []
            for g in range(mac):
                cps.append(pltpu.async_copy(
                    rows_v.at[g], agg_sh.at[idx2_v.at[g]], sem, add=True))
                cps.append(pltpu.async_copy(
                    ones_v, cnt_sh.at[idx2_v.at[g]], sem, add=True))
            for c in cps:
                c.wait()
            return carry

        lax.fori_loop(0, nmac, step, 0)
        plsc.subcore_barrier()
        pltpu.sync_copy(agg_sh.at[pl.ds(r0, rpt)],
                        agg_out.at[pl.ds(cid * h1 + r0, rpt)])
        pltpu.sync_copy(cnt_sh.at[pl.ds(r0, rpt)],
                        cnt_out.at[pl.ds(cid * h1 + r0, rpt)])

    return k(msg3d, dst2d, zero_nd, zero_n, ones128)


def _tc_edge(ea_t, xjp, W1T, b1c, W2T, b2c, D, B):
    DE, Ep = ea_t.shape
    H = W1T.shape[0]
    DD = W2T.shape[0]
    Bp = B // 8
    nblk = Ep // B

    def body(ea_ref, xj_ref, w1_ref, b1_ref, w2_ref, b2_ref, out_ref):
        eaT = ea_ref[...]
        xpt = jnp.transpose(xj_ref[...])
        xjT = jnp.concatenate(
            [xpt[j * D:(j + 1) * D, :] for j in range(8)], axis=1)
        hidT = jnp.maximum(
            jnp.dot(w1_ref[...], eaT, preferred_element_type=jnp.float32)
            + b1_ref[...], 0.0)
        wT = jnp.dot(w2_ref[...], hidT,
                     preferred_element_type=jnp.float32) + b2_ref[...]
        xrep = jnp.broadcast_to(xjT[:, None, :], (D, D, B)).reshape(DD, B)
        msgT = jnp.sum((xrep * wT).reshape(D, D, B), axis=0)
        mstack = jnp.concatenate(
            [msgT[:, j * Bp:(j + 1) * Bp] for j in range(8)], axis=0)
        out_ref[...] = jnp.transpose(mstack)

    return pl.pallas_call(
        body,
        grid=(nblk,),
        in_specs=[
            pl.BlockSpec((DE, B), lambda i: (0, i)),
            pl.BlockSpec((Bp, 128), lambda i: (i, 0)),
            pl.BlockSpec((H, DE), lambda i: (0, 0)),
            pl.BlockSpec((H, 1), lambda i: (0, 0)),
            pl.BlockSpec((DD, H), lambda i: (0, 0)),
            pl.BlockSpec((DD, 1), lambda i: (0, 0)),
        ],
        out_specs=pl.BlockSpec((Bp, 128), lambda i: (i, 0)),
        out_shape=jax.ShapeDtypeStruct((Ep // 8, 128), jnp.float32),
    )(ea_t, xjp, W1T, b1c, W2T, b2c)


def _tc_final(xp, aggp, cntp, bd, rsel, msel, bias_t, gamma_t, beta_t, n_real):
    N8 = xp.shape[0]
    inv_n = 1.0 / float(n_real)

    def body(x_ref, a_ref, c_ref, bd_ref, r_ref, m_ref, bias_ref, g_ref,
             be_ref, o_ref):
        inv = 1.0 / jnp.maximum(c_ref[...], 1.0)
        inv128 = jnp.dot(inv, r_ref[...],
                         preferred_element_type=jnp.float32)
        h = (a_ref[...] * inv128
             + jnp.dot(x_ref[...], bd_ref[...],
                       preferred_element_type=jnp.float32)
             + bias_ref[...])
        s1 = jnp.sum(h, axis=0, keepdims=True)
        mean = jnp.dot(s1, m_ref[...],
                       preferred_element_type=jnp.float32) * inv_n
        d = h - mean
        s2 = jnp.sum(d * d, axis=0, keepdims=True)
        var = jnp.dot(s2, m_ref[...],
                      preferred_element_type=jnp.float32) * inv_n
        hn = d * lax.rsqrt(var + _EPS) * g_ref[...] + be_ref[...]
        o_ref[...] = x_ref[...] + jnp.maximum(hn, 0.0)

    return pl.pallas_call(
        body,
        out_shape=jax.ShapeDtypeStruct((N8, 128), jnp.float32),
    )(xp, aggp, cntp, bd, rsel, msel, bias_t, gamma_t, beta_t)


def kernel(x, edge_index, edge_attr, W1, b1, W2, b2, root, bias, gamma, beta):
    Nn, D = x.shape
    E, DE = edge_attr.shape
    H = W1.shape[1]

    src = edge_index[0].astype(jnp.int32)
    dst = edge_index[1].astype(jnp.int32)

    G = math.ceil(E / 128)
    G = math.ceil(G / (8 * _NW)) * (8 * _NW)
    Ep = G * 128
    pad = Ep - E
    src_p = jnp.concatenate([src, jnp.zeros((pad,), jnp.int32)]).reshape(G, 128)
    dst_p = jnp.concatenate(
        [dst, jnp.full((pad,), Nn, jnp.int32)]).reshape(G, 128)
    h1 = math.ceil((Nn + 1) / 256) * 128
    a1 = h1 + 128
    B = 2048
    Bp = B // 8
    ea_t = jnp.concatenate(
        [edge_attr.T, jnp.zeros((DE, pad), jnp.float32)], axis=1)
    ea_t = jnp.stack(
        [ea_t[:, j::8].reshape(DE, Ep // B, Bp) for j in range(8)],
        axis=2).reshape(DE, Ep)

    xj3 = _sc_gather(x, src_p)

    xjp = xj3.reshape(Ep // 8, 128)
    msgp = _tc_edge(ea_t, xjp, W1.T, b1.reshape(H, 1),
                    W2.T, b2.reshape(D * D, 1), D, B)

    zero_nd = jnp.zeros((a1, D), jnp.float32)
    zero_n = jnp.zeros((a1,), jnp.float32)
    ones128 = jnp.ones((128,), jnp.float32)
    agg_o, cnt_o = _sc_scatter(msgp.reshape(G, 128, D), dst_p,
                               zero_nd, zero_n, ones128, h1, a1)

    P = 128 // D
    N8 = Nn // P
    aggp = agg_o.reshape(2 * h1 * D // 128, 128)[:N8]
    cntp = cnt_o.reshape(2 * h1 // P, P)[:N8]
    bd = jnp.kron(jnp.eye(P, dtype=jnp.float32), root)
    rsel = jnp.kron(jnp.eye(P, dtype=jnp.float32),
                    jnp.ones((1, D), jnp.float32))
    msel = jnp.kron(jnp.ones((P, P), jnp.float32),
                    jnp.eye(D, dtype=jnp.float32))
    outp = _tc_final(x.reshape(N8, 128), aggp, cntp, bd, rsel, msel,
                     jnp.tile(bias, P).reshape(1, 128),
                     jnp.tile(gamma, P).reshape(1, 128),
                     jnp.tile(beta, P).reshape(1, 128), Nn)
    return outp.reshape(Nn, D)


# speedup vs baseline: 4.4248x; 1.2023x over previous
import functools
import math

import jax
import jax.numpy as jnp
from jax import lax
from jax.experimental import pallas as pl
from jax.experimental.pallas import tpu as pltpu
from jax.experimental.pallas import tpu_sc as plsc

_EPS = 1e-5
_NC = 2
_NS = 16
_NW = _NC * _NS


def _pick_mac(gpw):
    for m in (40, 32, 24, 16, 8):
        if gpw % m == 0:
            return m
    return 8


def _sc_gather(x, src2d):
    G = src2d.shape[0]
    D = x.shape[1]
    gpw = G // _NW
    mac = _pick_mac(gpw)
    nmac = gpw // mac
    mesh = plsc.VectorSubcoreMesh(core_axis_name="c", subcore_axis_name="s")

    @functools.partial(
        pl.kernel,
        mesh=mesh,
        out_type=jax.ShapeDtypeStruct((G, 128, D), jnp.float32),
        scratch_types=[
            pltpu.VMEM((mac, 128), jnp.int32),
            pltpu.VMEM((mac, 128, D), jnp.float32),
            pltpu.SemaphoreType.DMA,
        ],
        compiler_params=pltpu.CompilerParams(use_tc_tiling_on_sc=False),
    )
    def k(x_hbm, src_hbm, out_hbm, idx_v, rows_v, sem):
        wid = lax.axis_index("s") * _NC + lax.axis_index("c")
        g0 = wid * gpw
        for m in range(nmac):
            gb = g0 + m * mac
            pltpu.sync_copy(src_hbm.at[pl.ds(gb, mac)], idx_v)
            cps = [
                pltpu.async_copy(x_hbm.at[idx_v.at[r]], rows_v.at[r], sem)
                for r in range(mac)
            ]
            for c in cps:
                c.wait()
            pltpu.sync_copy(rows_v, out_hbm.at[pl.ds(gb, mac)])

    return k(x, src2d)


def _sc_scatter(msg3d, dst2d, zero_nd, zero_n, ones128, h1, a1):
    G = dst2d.shape[0]
    D = msg3d.shape[2]
    mac = 16
    nmac = G // (_NS * mac)
    rpt_z = a1 // _NS
    rpt = h1 // _NS
    trash = a1 - 1
    mesh = plsc.VectorSubcoreMesh(core_axis_name="c", subcore_axis_name="s")

    @functools.partial(
        pl.kernel,
        mesh=mesh,
        out_type=(
            jax.ShapeDtypeStruct((_NC * h1, D), jnp.float32),
            jax.ShapeDtypeStruct((_NC * h1,), jnp.float32),
        ),
        scratch_types=[
            pltpu.VMEM((mac, 128), jnp.int32),
            pltpu.VMEM((mac, 128), jnp.int32),
            pltpu.VMEM((mac, 128, D), jnp.float32),
            pltpu.VMEM((128,), jnp.float32),
            pltpu.VMEM_SHARED((a1, D), jnp.float32),
            pltpu.VMEM_SHARED((a1,), jnp.float32),
            pltpu.SemaphoreType.DMA,
        ],
        compiler_params=pltpu.CompilerParams(use_tc_tiling_on_sc=False),
    )
    def k(msg_hbm, dst_hbm, z2_hbm, z1_hbm, one_hbm, agg_out, cnt_out,
          idx_v, idx2_v, rows_v, ones_v, agg_sh, cnt_sh, sem):
        cid = lax.axis_index("c")
        sid = lax.axis_index("s")
        rz0 = sid * rpt_z
        r0 = sid * rpt
        lo_bound = cid * h1
        pltpu.sync_copy(z2_hbm.at[pl.ds(rz0, rpt_z)],
                        agg_sh.at[pl.ds(rz0, rpt_z)])
        pltpu.sync_copy(z1_hbm.at[pl.ds(rz0, rpt_z)],
                        cnt_sh.at[pl.ds(rz0, rpt_z)])
        pltpu.sync_copy(one_hbm, ones_v)
        plsc.subcore_barrier()

        def step(m, carry):
            gb = pl.multiple_of((sid * nmac + m) * mac, 8)
            pltpu.sync_copy(dst_hbm.at[pl.ds(gb, mac)], idx_v)
            pltpu.sync_copy(msg_hbm.at[pl.ds(gb, mac)], rows_v)
            for g in range(mac):
                for c16 in range(8):
                    d = idx_v[g, pl.ds(c16 * 16, 16)]
                    loc = d - lo_bound
                    ok = (loc >= 0) & (loc < h1)
                    idx2_v[g, pl.ds(c16 * 16, 16)] = jnp.where(ok, loc, trash)
            cps = []
            for g in range(mac):
                cps.append(pltpu.async_copy(
                    rows_v.at[g], agg_sh.at[idx2_v.at[g]], sem, add=True))
                cps.append(pltpu.async_copy(
                    ones_v, cnt_sh.at[idx2_v.at[g]], sem, add=True))
            for c in cps:
                c.wait()
            return carry

        lax.fori_loop(0, nmac, step, 0)
        plsc.subcore_barrier()
        pltpu.sync_copy(agg_sh.at[pl.ds(r0, rpt)],
                        agg_out.at[pl.ds(cid * h1 + r0, rpt)])
        pltpu.sync_copy(cnt_sh.at[pl.ds(r0, rpt)],
                        cnt_out.at[pl.ds(cid * h1 + r0, rpt)])

    return k(msg3d, dst2d, zero_nd, zero_n, ones128)


def _tc_edge(ea_t, xjp, W1T, b1c, W2T, b2c, D, B):
    DE, Ep = ea_t.shape
    H = W1T.shape[0]
    DD = W2T.shape[0]
    Bp = B // 8
    nblk = Ep // B

    def body(ea_ref, xj_ref, w1_ref, b1_ref, w2_ref, b2_ref, out_ref):
        eaT = ea_ref[...]
        xpt = jnp.transpose(xj_ref[...])
        xjT = jnp.concatenate(
            [xpt[j * D:(j + 1) * D, :] for j in range(8)], axis=1)
        hidT = jnp.maximum(
            jnp.dot(w1_ref[...], eaT, preferred_element_type=jnp.float32)
            + b1_ref[...], 0.0)
        wT = jnp.dot(w2_ref[...], hidT,
                     preferred_element_type=jnp.float32) + b2_ref[...]
        xrep = jnp.broadcast_to(xjT[:, None, :], (D, D, B)).reshape(DD, B)
        msgT = jnp.sum((xrep * wT).reshape(D, D, B), axis=0)
        mstack = jnp.concatenate(
            [msgT[:, j * Bp:(j + 1) * Bp] for j in range(8)], axis=0)
        out_ref[...] = jnp.transpose(mstack)

    return pl.pallas_call(
        body,
        grid=(nblk,),
        in_specs=[
            pl.BlockSpec((DE, B), lambda i: (0, i)),
            pl.BlockSpec((Bp, 128), lambda i: (i, 0)),
            pl.BlockSpec((H, DE), lambda i: (0, 0)),
            pl.BlockSpec((H, 1), lambda i: (0, 0)),
            pl.BlockSpec((DD, H), lambda i: (0, 0)),
            pl.BlockSpec((DD, 1), lambda i: (0, 0)),
        ],
        out_specs=pl.BlockSpec((Bp, 128), lambda i: (i, 0)),
        out_shape=jax.ShapeDtypeStruct((Ep // 8, 128), jnp.float32),
    )(ea_t, xjp, W1T, b1c, W2T, b2c)


def _tc_final(xp, aggp, aggp2, cntp, cntp2, bd, rsel, msel, bias_t, gamma_t,
              beta_t, n_real):
    N8 = xp.shape[0]
    inv_n = 1.0 / float(n_real)

    def body(x_ref, a_ref, a2_ref, c_ref, c2_ref, bd_ref, r_ref, m_ref,
             bias_ref, g_ref, be_ref, o_ref):
        inv = 1.0 / jnp.maximum(c_ref[...] + c2_ref[...], 1.0)
        inv128 = jnp.dot(inv, r_ref[...],
                         preferred_element_type=jnp.float32)
        h = ((a_ref[...] + a2_ref[...]) * inv128
             + jnp.dot(x_ref[...], bd_ref[...],
                       preferred_element_type=jnp.float32)
             + bias_ref[...])
        s1 = jnp.sum(h, axis=0, keepdims=True)
        mean = jnp.dot(s1, m_ref[...],
                       preferred_element_type=jnp.float32) * inv_n
        d = h - mean
        s2 = jnp.sum(d * d, axis=0, keepdims=True)
        var = jnp.dot(s2, m_ref[...],
                      preferred_element_type=jnp.float32) * inv_n
        hn = d * lax.rsqrt(var + _EPS) * g_ref[...] + be_ref[...]
        o_ref[...] = x_ref[...] + jnp.maximum(hn, 0.0)

    return pl.pallas_call(
        body,
        out_shape=jax.ShapeDtypeStruct((N8, 128), jnp.float32),
    )(xp, aggp, aggp2, cntp, cntp2, bd, rsel, msel, bias_t, gamma_t, beta_t)


def kernel(x, edge_index, edge_attr, W1, b1, W2, b2, root, bias, gamma, beta):
    Nn, D = x.shape
    E, DE = edge_attr.shape
    H = W1.shape[1]

    src = edge_index[0].astype(jnp.int32)
    dst = edge_index[1].astype(jnp.int32)

    G = math.ceil(E / 128)
    G = math.ceil(G / (8 * _NW)) * (8 * _NW)
    Ep = G * 128
    pad = Ep - E
    src_p = jnp.concatenate([src, jnp.zeros((pad,), jnp.int32)]).reshape(G, 128)
    dst_p = jnp.concatenate(
        [dst, jnp.full((pad,), Nn, jnp.int32)]).reshape(G, 128)
    h1 = math.ceil((Nn + 1) / 256) * 128
    a1 = h1 + 128
    B = 2048
    Bp = B // 8
    ea_t = jnp.concatenate(
        [edge_attr.T, jnp.zeros((DE, pad), jnp.float32)], axis=1)
    ea_t = jnp.stack(
        [ea_t[:, j::8].reshape(DE, Ep // B, Bp) for j in range(8)],
        axis=2).reshape(DE, Ep)

    Ga = (G // 2 // 256) * 256
    Epa = Ga * 128
    zero_nd = jnp.zeros((a1, D), jnp.float32)
    zero_n = jnp.zeros((a1,), jnp.float32)
    ones128 = jnp.ones((128,), jnp.float32)
    W1T = W1.T
    b1c = b1.reshape(H, 1)
    W2T = W2.T
    b2c = b2.reshape(D * D, 1)

    xj3_a = _sc_gather(x, src_p[:Ga])
    xj3_b = _sc_gather(x, src_p[Ga:])
    msgp_a = _tc_edge(ea_t[:, :Epa], xj3_a.reshape(Epa // 8, 128),
                      W1T, b1c, W2T, b2c, D, B)
    agg_a, cnt_a = _sc_scatter(msgp_a.reshape(Ga, 128, D), dst_p[:Ga],
                               zero_nd, zero_n, ones128, h1, a1)
    msgp_b = _tc_edge(ea_t[:, Epa:], xj3_b.reshape((Ep - Epa) // 8, 128),
                      W1T, b1c, W2T, b2c, D, B)
    agg_b, cnt_b = _sc_scatter(msgp_b.reshape(G - Ga, 128, D), dst_p[Ga:],
                               zero_nd, zero_n, ones128, h1, a1)

    P = 128 // D
    N8 = Nn // P
    aggp = agg_a.reshape(2 * h1 * D // 128, 128)[:N8]
    cntp = cnt_a.reshape(2 * h1 // P, P)[:N8]
    aggp2 = agg_b.reshape(2 * h1 * D // 128, 128)[:N8]
    cntp2 = cnt_b.reshape(2 * h1 // P, P)[:N8]
    bd = jnp.kron(jnp.eye(P, dtype=jnp.float32), root)
    rsel = jnp.kron(jnp.eye(P, dtype=jnp.float32),
                    jnp.ones((1, D), jnp.float32))
    msel = jnp.kron(jnp.ones((P, P), jnp.float32),
                    jnp.eye(D, dtype=jnp.float32))
    outp = _tc_final(x.reshape(N8, 128), aggp, aggp2, cntp, cntp2,
                     bd, rsel, msel,
                     jnp.tile(bias, P).reshape(1, 128),
                     jnp.tile(gamma, P).reshape(1, 128),
                     jnp.tile(beta, P).reshape(1, 128), Nn)
    return outp.reshape(Nn, D)
